# baseline (speedup 1.0000x reference)
import math

import jax
import jax.numpy as jnp
from jax.experimental import pallas as pl
from jax.experimental.pallas import tpu as pltpu

_N_VOCAB = 1000000
_N_DIM = 64
_SIGMA = 1.0

_BLOCK_COLS = 49920
_NUM_BLOCKS = -(-_N_VOCAB // _BLOCK_COLS)
_TAIL_COLS = _N_VOCAB - (_NUM_BLOCKS - 1) * _BLOCK_COLS

_N_ELEMS = 2 * _N_VOCAB * _N_DIM
_CONST = _N_ELEMS * (-math.log(_SIGMA) - 0.5 * math.log(2.0 * math.pi))


def _accumulate(acc, w, c):
    for k in range(_BLOCK_COLS // 128):
        ws = w[:, k * 128 : (k + 1) * 128]
        cs = c[:, k * 128 : (k + 1) * 128]
        acc = acc + ws * ws + cs * cs
    return acc


def _prior_body(w_ref, c_ref, o_ref, acc_ref):
    i = pl.program_id(0)

    @pl.when(i == 0)
    def _init():
        acc_ref[...] = jnp.zeros((_N_DIM, 128), jnp.float32)

    @pl.when(i < _NUM_BLOCKS - 1)
    def _full_block():
        acc_ref[...] = _accumulate(acc_ref[...], w_ref[...], c_ref[...])

    @pl.when(i == _NUM_BLOCKS - 1)
    def _ragged_block_and_finalize():
        lane = jax.lax.broadcasted_iota(jnp.int32, (_N_DIM, 128), 1)
        acc = acc_ref[...]
        for k in range(-(-_TAIL_COLS // 128)):
            sl = slice(k * 128, (k + 1) * 128)
            ws = w_ref[:, sl]
            cs = c_ref[:, sl]
            valid = _TAIL_COLS - k * 128
            if valid < 128:
                m = lane < valid
                ws = jnp.where(m, ws, 0.0)
                cs = jnp.where(m, cs, 0.0)
            acc = acc + ws * ws + cs * cs
        o_ref[0, 0] = -0.5 * jnp.sum(acc) + _CONST


def kernel(target_ixs, context_ixs, negative_sample_ixs, word_weight, context_weight):
    del target_ixs, context_ixs, negative_sample_ixs
    w = word_weight.T
    c = context_weight.T

    out = pl.pallas_call(
        _prior_body,
        grid=(_NUM_BLOCKS,),
        in_specs=[
            pl.BlockSpec((_N_DIM, _BLOCK_COLS), lambda i: (0, i)),
            pl.BlockSpec((_N_DIM, _BLOCK_COLS), lambda i: (0, i)),
        ],
        out_specs=pl.BlockSpec(
            (1, 1), lambda i: (0, 0), memory_space=pltpu.MemorySpace.SMEM
        ),
        out_shape=jax.ShapeDtypeStruct((1, 1), jnp.float32),
        scratch_shapes=[pltpu.VMEM((_N_DIM, 128), jnp.float32)],
    )(w, c)
    return out[0, 0]


# speedup vs baseline: 1.0125x; 1.0125x over previous
import math

import jax
import jax.numpy as jnp
from jax.experimental import pallas as pl
from jax.experimental.pallas import tpu as pltpu

_N_VOCAB = 1000000
_N_DIM = 64
_SIGMA = 1.0

_BLOCK_COLS = 57344
_NUM_BLOCKS = -(-_N_VOCAB // _BLOCK_COLS)
_TAIL_COLS = _N_VOCAB - (_NUM_BLOCKS - 1) * _BLOCK_COLS

_N_ELEMS = 2 * _N_VOCAB * _N_DIM
_CONST = _N_ELEMS * (-math.log(_SIGMA) - 0.5 * math.log(2.0 * math.pi))


def _accumulate(acc, w, c):
    for k in range(_BLOCK_COLS // 128):
        ws = w[:, k * 128 : (k + 1) * 128]
        cs = c[:, k * 128 : (k + 1) * 128]
        acc = acc + ws * ws + cs * cs
    return acc


def _prior_body(w_ref, c_ref, o_ref, acc_ref):
    i = pl.program_id(0)

    @pl.when(i == 0)
    def _init():
        acc_ref[...] = jnp.zeros((_N_DIM, 128), jnp.float32)

    @pl.when(i < _NUM_BLOCKS - 1)
    def _full_block():
        acc_ref[...] = _accumulate(acc_ref[...], w_ref[...], c_ref[...])

    @pl.when(i == _NUM_BLOCKS - 1)
    def _ragged_block_and_finalize():
        lane = jax.lax.broadcasted_iota(jnp.int32, (_N_DIM, 128), 1)
        acc = acc_ref[...]
        for k in range(-(-_TAIL_COLS // 128)):
            sl = slice(k * 128, (k + 1) * 128)
            ws = w_ref[:, sl]
            cs = c_ref[:, sl]
            valid = _TAIL_COLS - k * 128
            if valid < 128:
                m = lane < valid
                ws = jnp.where(m, ws, 0.0)
                cs = jnp.where(m, cs, 0.0)
            acc = acc + ws * ws + cs * cs
        o_ref[0, 0] = -0.5 * jnp.sum(acc) + _CONST


def kernel(target_ixs, context_ixs, negative_sample_ixs, word_weight, context_weight):
    del target_ixs, context_ixs, negative_sample_ixs
    w = word_weight.T
    c = context_weight.T

    out = pl.pallas_call(
        _prior_body,
        grid=(_NUM_BLOCKS,),
        in_specs=[
            pl.BlockSpec((_N_DIM, _BLOCK_COLS), lambda i: (0, i)),
            pl.BlockSpec((_N_DIM, _BLOCK_COLS), lambda i: (0, i)),
        ],
        out_specs=pl.BlockSpec(
            (1, 1), lambda i: (0, 0), memory_space=pltpu.MemorySpace.SMEM
        ),
        out_shape=jax.ShapeDtypeStruct((1, 1), jnp.float32),
        scratch_shapes=[pltpu.VMEM((_N_DIM, 128), jnp.float32)],
    )(w, c)
    return out[0, 0]


# speedup vs baseline: 1.0153x; 1.0028x over previous
import math

import jax
import jax.numpy as jnp
from jax.experimental import pallas as pl
from jax.experimental.pallas import tpu as pltpu

_N_VOCAB = 1000000
_N_DIM = 64
_SIGMA = 1.0

_BLOCK_COLS = 49152
_NUM_BLOCKS = -(-_N_VOCAB // _BLOCK_COLS)
_TAIL_COLS = _N_VOCAB - (_NUM_BLOCKS - 1) * _BLOCK_COLS

_N_ELEMS = 2 * _N_VOCAB * _N_DIM
_CONST = _N_ELEMS * (-math.log(_SIGMA) - 0.5 * math.log(2.0 * math.pi))


def _accumulate(acc, w, c):
    for k in range(_BLOCK_COLS // 128):
        ws = w[:, k * 128 : (k + 1) * 128]
        cs = c[:, k * 128 : (k + 1) * 128]
        acc = acc + ws * ws + cs * cs
    return acc


def _prior_body(w_ref, c_ref, o_ref, acc_ref):
    i = pl.program_id(0)

    @pl.when(i == 0)
    def _init():
        acc_ref[...] = jnp.zeros((_N_DIM, 128), jnp.float32)

    @pl.when(i < _NUM_BLOCKS - 1)
    def _full_block():
        acc_ref[...] = _accumulate(acc_ref[...], w_ref[...], c_ref[...])

    @pl.when(i == _NUM_BLOCKS - 1)
    def _ragged_block_and_finalize():
        lane = jax.lax.broadcasted_iota(jnp.int32, (_N_DIM, 128), 1)
        acc = acc_ref[...]
        for k in range(-(-_TAIL_COLS // 128)):
            sl = slice(k * 128, (k + 1) * 128)
            ws = w_ref[:, sl]
            cs = c_ref[:, sl]
            valid = _TAIL_COLS - k * 128
            if valid < 128:
                m = lane < valid
                ws = jnp.where(m, ws, 0.0)
                cs = jnp.where(m, cs, 0.0)
            acc = acc + ws * ws + cs * cs
        o_ref[0, 0] = -0.5 * jnp.sum(acc) + _CONST


def kernel(target_ixs, context_ixs, negative_sample_ixs, word_weight, context_weight):
    del target_ixs, context_ixs, negative_sample_ixs
    w = word_weight.T
    c = context_weight.T

    out = pl.pallas_call(
        _prior_body,
        grid=(_NUM_BLOCKS,),
        in_specs=[
            pl.BlockSpec((_N_DIM, _BLOCK_COLS), lambda i: (0, i)),
            pl.BlockSpec((_N_DIM, _BLOCK_COLS), lambda i: (0, i)),
        ],
        out_specs=pl.BlockSpec(
            (1, 1), lambda i: (0, 0), memory_space=pltpu.MemorySpace.SMEM
        ),
        out_shape=jax.ShapeDtypeStruct((1, 1), jnp.float32),
        scratch_shapes=[pltpu.VMEM((_N_DIM, 128), jnp.float32)],
    )(w, c)
    return out[0, 0]
